# initial kernel scaffold (unmeasured)
import jax
import jax.numpy as jnp
from jax import lax
from jax.experimental import pallas as pl
from jax.experimental.pallas import tpu as pltpu

B, S, H_LOC, D = 4, 1024, 16, 128
K = H_LOC * D
N = 4096
S_HALF = S // 2


def kernel(O, Wo):
    x = O.reshape(B * S, K).astype(jnp.bfloat16)
    w = Wo.astype(jnp.bfloat16)

    def body(x_ref, w_ref, out_ref, send_buf, recv_buf, send_sems, recv_sems):
        my_x = lax.axis_index("x")
        my_y = lax.axis_index("y")
        my_z = lax.axis_index("z")
        peer = (1 - my_x, my_y, my_z)

        barrier_sem = pltpu.get_barrier_semaphore()
        pl.semaphore_signal(
            barrier_sem, inc=1, device_id=peer,
            device_id_type=pl.DeviceIdType.MESH,
        )
        pl.semaphore_wait(barrier_sem, 1)

        rdmas = []
        for b in range(B):
            theirs = b * S + (1 - my_x) * S_HALF
            part = jnp.dot(
                x_ref[pl.ds(theirs, S_HALF), :], w_ref[:, :],
                preferred_element_type=jnp.float32,
            )
            send_buf[b] = part.astype(jnp.bfloat16)
            rdma = pltpu.make_async_remote_copy(
                src_ref=send_buf.at[b],
                dst_ref=recv_buf.at[b],
                send_sem=send_sems.at[b],
                recv_sem=recv_sems.at[b],
                device_id=peer,
                device_id_type=pl.DeviceIdType.MESH,
            )
            rdma.start()
            rdmas.append(rdma)

            mine = b * S + my_x * S_HALF
            out_ref[b] = jnp.dot(
                x_ref[pl.ds(mine, S_HALF), :], w_ref[:, :],
                preferred_element_type=jnp.float32,
            )

        for b in range(B):
            rdmas[b].wait_recv()
            out_ref[b] = out_ref[b] + recv_buf[b].astype(jnp.float32)
            rdmas[b].wait_send()

    return pl.pallas_call(
        body,
        out_shape=jax.ShapeDtypeStruct((B, S_HALF, N), jnp.float32),
        in_specs=[
            pl.BlockSpec(memory_space=pltpu.VMEM),
            pl.BlockSpec(memory_space=pltpu.VMEM),
        ],
        out_specs=pl.BlockSpec(memory_space=pltpu.VMEM),
        scratch_shapes=[
            pltpu.VMEM((B, S_HALF, N), jnp.bfloat16),
            pltpu.VMEM((B, S_HALF, N), jnp.bfloat16),
            pltpu.SemaphoreType.DMA((B,)),
            pltpu.SemaphoreType.DMA((B,)),
        ],
        compiler_params=pltpu.CompilerParams(collective_id=0),
    )(x, w)


# baseline (device time: 300712 ns/iter reference)
import jax
import jax.numpy as jnp
from jax import lax
from jax.experimental import pallas as pl
from jax.experimental.pallas import tpu as pltpu

B, S, H_LOC, D = 4, 1024, 16, 128
K = H_LOC * D
N = 4096
S_HALF = S // 2
N_CHUNK = 2048


def kernel(O, Wo):
    x = O.reshape(B * S, K).astype(jnp.bfloat16)
    w = Wo.astype(jnp.bfloat16)

    def body(x_hbm, w_ref, out_ref, x_tiles, send_buf, recv_buf,
             x_sems, send_sems, recv_sems, credit_sem):
        my_x = lax.axis_index("x")
        my_y = lax.axis_index("y")
        my_z = lax.axis_index("z")
        peer = (1 - my_x, my_y, my_z)

        barrier_sem = pltpu.get_barrier_semaphore()
        pl.semaphore_signal(
            barrier_sem, inc=1, device_id=peer,
            device_id_type=pl.DeviceIdType.MESH,
        )
        pl.semaphore_wait(barrier_sem, 1)

        def x_load(b):
            return pltpu.make_async_copy(
                x_hbm.at[pl.ds(b * S, S), :], x_tiles.at[b % 2],
                x_sems.at[b % 2],
            )

        x_load(0).start()
        rdmas = [None, None]
        for b in range(B):
            slot = b % 2
            if b + 1 < B:
                x_load(b + 1).start()
            x_load(b).wait()

            if b >= 2:
                rdmas[slot].wait_send()
            theirs = b * S
            for nj in range(N // N_CHUNK):
                cols = slice(nj * N_CHUNK, (nj + 1) * N_CHUNK)
                send_buf[slot, :, cols] = jnp.dot(
                    x_tiles[slot, pl.ds((1 - my_x) * S_HALF, S_HALF), :],
                    w_ref[:, cols],
                    preferred_element_type=jnp.float32,
                ).astype(jnp.bfloat16)

            if b >= 2:
                pl.semaphore_wait(credit_sem, 1)
            rdma = pltpu.make_async_remote_copy(
                src_ref=send_buf.at[slot],
                dst_ref=recv_buf.at[slot],
                send_sem=send_sems.at[slot],
                recv_sem=recv_sems.at[slot],
                device_id=peer,
                device_id_type=pl.DeviceIdType.MESH,
            )
            rdma.start()
            rdmas[slot] = rdma

            for nj in range(N // N_CHUNK):
                cols = slice(nj * N_CHUNK, (nj + 1) * N_CHUNK)
                out_ref[b, :, cols] = jnp.dot(
                    x_tiles[slot, pl.ds(my_x * S_HALF, S_HALF), :],
                    w_ref[:, cols],
                    preferred_element_type=jnp.float32,
                ).astype(jnp.bfloat16)

            rdma.wait_recv()
            out_ref[b] = out_ref[b] + recv_buf[slot]
            if b < B - 2:
                pl.semaphore_signal(
                    credit_sem, inc=1, device_id=peer,
                    device_id_type=pl.DeviceIdType.MESH,
                )

        rdmas[(B - 2) % 2].wait_send()
        rdmas[(B - 1) % 2].wait_send()

    out = pl.pallas_call(
        body,
        out_shape=jax.ShapeDtypeStruct((B, S_HALF, N), jnp.bfloat16),
        in_specs=[
            pl.BlockSpec(memory_space=pltpu.MemorySpace.HBM),
            pl.BlockSpec(memory_space=pltpu.VMEM),
        ],
        out_specs=pl.BlockSpec(memory_space=pltpu.VMEM),
        scratch_shapes=[
            pltpu.VMEM((2, S, K), jnp.bfloat16),
            pltpu.VMEM((2, S_HALF, N), jnp.bfloat16),
            pltpu.VMEM((2, S_HALF, N), jnp.bfloat16),
            pltpu.SemaphoreType.DMA((2,)),
            pltpu.SemaphoreType.DMA((2,)),
            pltpu.SemaphoreType.DMA((2,)),
            pltpu.SemaphoreType.REGULAR,
        ],
        compiler_params=pltpu.CompilerParams(
            collective_id=0,
            vmem_limit_bytes=100 * 1024 * 1024,
        ),
    )(x, w)
    return out.astype(jnp.float32)


# device time: 264374 ns/iter; 1.1374x vs baseline; 1.1374x over previous
import jax
import jax.numpy as jnp
from jax import lax
from jax.experimental import pallas as pl
from jax.experimental.pallas import tpu as pltpu

B, S, H_LOC, D = 4, 1024, 16, 128
K = H_LOC * D
N = 4096
S_HALF = S // 2
N_CHUNK = 2048


def kernel(O, Wo):
    x = O.reshape(B * S, K).astype(jnp.bfloat16)
    w = Wo.astype(jnp.bfloat16)

    def body(x_hbm, w_ref, out_ref, x_tiles, send_buf, recv_buf,
             x_sems, send_sems, recv_sems, credit_sem):
        my_x = lax.axis_index("x")
        my_y = lax.axis_index("y")
        my_z = lax.axis_index("z")
        peer = (1 - my_x, my_y, my_z)

        barrier_sem = pltpu.get_barrier_semaphore()
        pl.semaphore_signal(
            barrier_sem, inc=1, device_id=peer,
            device_id_type=pl.DeviceIdType.MESH,
        )
        pl.semaphore_wait(barrier_sem, 1)

        def x_load(b):
            return pltpu.make_async_copy(
                x_hbm.at[pl.ds(b * S, S), :], x_tiles.at[b % 2],
                x_sems.at[b % 2],
            )

        x_load(0).start()
        rdmas = [None] * B
        for b in range(B):
            slot = b % 2
            if b + 1 < B:
                x_load(b + 1).start()
            x_load(b).wait()

            if b >= 2:
                rdmas[b - 2].wait_send()
            for nj in range(N // N_CHUNK):
                cols = slice(nj * N_CHUNK, (nj + 1) * N_CHUNK)
                send_buf[slot, :, cols] = jnp.dot(
                    x_tiles[slot, pl.ds((1 - my_x) * S_HALF, S_HALF), :],
                    w_ref[:, cols],
                    preferred_element_type=jnp.float32,
                ).astype(jnp.bfloat16)

            if b >= 2:
                pl.semaphore_wait(credit_sem, 1)
            rdma = pltpu.make_async_remote_copy(
                src_ref=send_buf.at[slot],
                dst_ref=recv_buf.at[slot],
                send_sem=send_sems.at[slot],
                recv_sem=recv_sems.at[slot],
                device_id=peer,
                device_id_type=pl.DeviceIdType.MESH,
            )
            rdma.start()
            rdmas[b] = rdma

            for nj in range(N // N_CHUNK):
                cols = slice(nj * N_CHUNK, (nj + 1) * N_CHUNK)
                out_ref[b, :, cols] = jnp.dot(
                    x_tiles[slot, pl.ds(my_x * S_HALF, S_HALF), :],
                    w_ref[:, cols],
                    preferred_element_type=jnp.float32,
                ).astype(jnp.bfloat16)

            if b > 0:
                rdmas[b - 1].wait_recv()
                out_ref[b - 1] = out_ref[b - 1] + recv_buf[(b - 1) % 2]
                if b - 1 < B - 2:
                    pl.semaphore_signal(
                        credit_sem, inc=1, device_id=peer,
                        device_id_type=pl.DeviceIdType.MESH,
                    )

        rdmas[B - 1].wait_recv()
        out_ref[B - 1] = out_ref[B - 1] + recv_buf[(B - 1) % 2]

        rdmas[B - 2].wait_send()
        rdmas[B - 1].wait_send()

    out = pl.pallas_call(
        body,
        out_shape=jax.ShapeDtypeStruct((B, S_HALF, N), jnp.bfloat16),
        in_specs=[
            pl.BlockSpec(memory_space=pltpu.MemorySpace.HBM),
            pl.BlockSpec(memory_space=pltpu.VMEM),
        ],
        out_specs=pl.BlockSpec(memory_space=pltpu.VMEM),
        scratch_shapes=[
            pltpu.VMEM((2, S, K), jnp.bfloat16),
            pltpu.VMEM((2, S_HALF, N), jnp.bfloat16),
            pltpu.VMEM((2, S_HALF, N), jnp.bfloat16),
            pltpu.SemaphoreType.DMA((2,)),
            pltpu.SemaphoreType.DMA((2,)),
            pltpu.SemaphoreType.DMA((2,)),
            pltpu.SemaphoreType.REGULAR,
        ],
        compiler_params=pltpu.CompilerParams(
            collective_id=0,
            vmem_limit_bytes=100 * 1024 * 1024,
        ),
    )(x, w)
    return out.astype(jnp.float32)
